# baseline (speedup 1.0000x reference)
import jax
import jax.numpy as jnp
from jax.experimental import pallas as pl
from jax.experimental.pallas import tpu as pltpu

B, C, H, W = 8, 384, 56, 56
P1 = C // 2
N = B * H * W
RB = 3584
GRID = N // RB


def _body(thr_ref, bn1_ref, bn2_ref, x0_ref, x1_ref, o0_ref, o1_ref):
    thr = thr_ref[0, 0]
    c_idx = jax.lax.broadcasted_iota(jnp.int32, (1, C), 1)
    first = c_idx < P1
    bn1 = jnp.abs(bn1_ref[...])
    bn2 = jnp.abs(bn2_ref[...])
    keep0 = jnp.logical_or(first, bn1 > thr)
    take0 = jnp.logical_and(jnp.logical_not(first), bn1 < thr)
    keep1 = jnp.logical_or(first, bn2 > thr)
    take1 = jnp.logical_and(jnp.logical_not(first), bn2 < thr)
    x0 = x0_ref[...]
    x1 = x1_ref[...]
    zero = jnp.zeros_like(x0)
    o0_ref[...] = jnp.where(keep0, x0, jnp.where(take0, x1, zero))
    o1_ref[...] = jnp.where(keep1, x1, jnp.where(take1, x0, zero))


@jax.jit
def _run(x0, x1, bn1, bn2, thr):
    x0r = x0.transpose(0, 2, 3, 1).reshape(N, C)
    x1r = x1.transpose(0, 2, 3, 1).reshape(N, C)
    bn1r = bn1.reshape(1, C)
    bn2r = bn2.reshape(1, C)
    thr_arr = jnp.asarray(thr, jnp.float32).reshape(1, 1)
    data_spec = pl.BlockSpec((RB, C), lambda i: (i, 0))
    vec_spec = pl.BlockSpec((1, C), lambda i: (0, 0))
    thr_spec = pl.BlockSpec((1, 1), lambda i: (0, 0))
    o0, o1 = pl.pallas_call(
        _body,
        grid=(GRID,),
        in_specs=[thr_spec, vec_spec, vec_spec, data_spec, data_spec],
        out_specs=[data_spec, data_spec],
        out_shape=[
            jax.ShapeDtypeStruct((N, C), jnp.float32),
            jax.ShapeDtypeStruct((N, C), jnp.float32),
        ],
        compiler_params=pltpu.CompilerParams(
            dimension_semantics=("parallel",),
        ),
    )(thr_arr, bn1r, bn2r, x0r, x1r)
    o0 = o0.reshape(B, H, W, C).transpose(0, 3, 1, 2)
    o1 = o1.reshape(B, H, W, C).transpose(0, 3, 1, 2)
    return o0, o1


def kernel(x0, x1, bn1_weight, bn2_weight, bn_threshold):
    return _run(x0, x1, bn1_weight, bn2_weight, bn_threshold)
